# baseline (speedup 1.0000x reference)
import functools

import jax
import jax.numpy as jnp
from jax import lax
from jax.experimental import pallas as pl
from jax.experimental.pallas import tpu as pltpu
from jax.experimental.pallas import tpu_sc as plsc

_B, _C, _H, _W = 16, 768, 14, 14
_M, _R = 32, 64
_D = 1024
_H1, _H2 = 1024, 512
_HW = _H * _W

_NW = 32

_PPS = 8


def _pool_kernel(feat_ref, bbox_ref, pooled_ref, ctx_ref):
    for k in range(_PPS):
        f = feat_ref[k]
        bb = bbox_ref[k]
        x0 = jnp.minimum(bb[:, 0:1], bb[:, 2:3])
        x1 = jnp.maximum(bb[:, 0:1], bb[:, 2:3])
        y0 = jnp.minimum(bb[:, 1:2], bb[:, 3:4])
        y1 = jnp.maximum(bb[:, 1:2], bb[:, 3:4])
        p = lax.broadcasted_iota(jnp.int32, (1, _HW), 1)
        xw = ((p % _W).astype(jnp.float32) + 0.5) / _W
        yh = ((p // _W).astype(jnp.float32) + 0.5) / _H
        mask = ((xw >= x0) & (xw <= x1) & (yh >= y0) & (yh <= y1))
        cnt = jnp.maximum(jnp.sum(mask.astype(jnp.float32), axis=1,
                                  keepdims=True), 1.0)
        row = lax.broadcasted_iota(jnp.int32, (8, _HW), 0)
        mext = jnp.concatenate([mask, row == 0], axis=0).astype(jnp.bfloat16)
        pooled = lax.dot_general(mext, f, (((1,), (1,)), ((), ())),
                                 preferred_element_type=jnp.float32)
        pooled_ref[k] = pooled[:_M] / cnt
        ctx_ref[k] = pooled[_M:] * (1.0 / _HW)


_NCB = 4
_CB = _D // _NCB


def _dense_kernel(pooled_ref, ctx8_ref, wobj_ref, bobj_ref, wctx_ref, bctx_ref,
                  w1_ref, obj_ref, y_ref, z_ref, obj_scr, ctx_scr):
    i = pl.program_id(0)

    @pl.when(i == 0)
    def _():
        ctx_scr[...] = jnp.maximum(
            jnp.dot(ctx8_ref[:, 0, :], wctx_ref[...],
                    preferred_element_type=jnp.float32) + bctx_ref[...], 0.0)

    @pl.when(i < _NCB)
    def _():
        obj = jnp.maximum(
            jnp.dot(pooled_ref[...], wobj_ref[...],
                    preferred_element_type=jnp.float32) + bobj_ref[...], 0.0)
        obj_ref[...] = obj
        obj_scr[:, pl.ds(i * _CB, _CB)] = obj

    @pl.when(i >= _NCB)
    def _():
        y_ref[...] = jnp.dot(obj_scr[...], w1_ref[:_D],
                             preferred_element_type=jnp.float32)
        z_ref[...] = jnp.dot(ctx_scr[...], w1_ref[_D:],
                             preferred_element_type=jnp.float32)


_OUT = _B * _R
_OPW = _OUT // _NW


def _pair_gather_sum(y, idx):
    @functools.partial(
        pl.kernel,
        mesh=plsc.VectorSubcoreMesh(core_axis_name="c", subcore_axis_name="s"),
        out_type=jax.ShapeDtypeStruct((_OUT, _D), jnp.float32),
        scratch_types=[
            pltpu.VMEM((_OPW,), jnp.int32),
            pltpu.VMEM((_OPW,), jnp.int32),
            pltpu.VMEM((_OPW, _D), jnp.float32),
            pltpu.VMEM((_OPW, _D), jnp.float32),
            pltpu.SemaphoreType.DMA,
        ],
    )
    def k(table_hbm, idx_hbm, out_hbm, idx0_v, idx1_v, buf0, buf1, sem):
        wid = lax.axis_index("s") * 2 + lax.axis_index("c")
        base = wid * _OPW
        pltpu.sync_copy(idx_hbm.at[pl.ds(base, _OPW)], idx0_v)
        pltpu.sync_copy(idx_hbm.at[pl.ds(_OUT + base, _OPW)], idx1_v)
        off = (wid // (_NW // _B)) * _M
        for j in range(_OPW // 16):
            s = pl.ds(j * 16, 16)
            idx0_v[s] = idx0_v[s] + off
            idx1_v[s] = idx1_v[s] + off
        cp0 = pltpu.async_copy(table_hbm.at[idx0_v], buf0, sem)
        cp1 = pltpu.async_copy(table_hbm.at[idx1_v], buf1, sem)
        cp0.wait()
        cp1.wait()

        def body(r, carry):
            for c in range(_D // 16):
                s = pl.ds(c * 16, 16)
                buf0[r, s] = buf0[r, s] + buf1[r, s]
            return carry

        lax.fori_loop(0, _OPW, body, 0)
        pltpu.sync_copy(buf0, out_hbm.at[pl.ds(base, _OPW)])

    return k(y, idx)


_BPS = 8


def _mlp_kernel(g_ref, z_ref, nrel_ref, b1_ref, w2_ref, b2_ref, out_ref):
    i = pl.program_id(0)
    for k in range(_BPS):
        b = i * _BPS + k
        nr = nrel_ref[b]
        valid = (lax.broadcasted_iota(jnp.int32, (_R, 1), 0) < nr).astype(jnp.float32)
        rows = g_ref[pl.ds(k * _R, _R), :]
        zrow = z_ref[pl.ds(b, 1), :]
        h = jnp.maximum(valid * (0.5 * rows + zrow) + b1_ref[...], 0.0)
        out_ref[pl.ds(k * _R, _R), :] = jnp.maximum(
            jnp.dot(h, w2_ref[...], preferred_element_type=jnp.float32)
            + b2_ref[...], 0.0)


def kernel(frame_deep_features, bboxes, num_obj, obj_pairs, num_rel,
           W_obj, b_obj, W_ctx, b_ctx, W1, b1, W2, b2):
    feat = frame_deep_features.reshape(_B, _C, _HW).astype(jnp.bfloat16)
    pooled, ctx8 = pl.pallas_call(
        _pool_kernel,
        grid=(_B // _PPS,),
        in_specs=[pl.BlockSpec((_PPS, _C, _HW), lambda b: (b, 0, 0)),
                  pl.BlockSpec((_PPS, _M, 4), lambda b: (b, 0, 0))],
        out_specs=[pl.BlockSpec((_PPS, _M, _C), lambda b: (b, 0, 0)),
                   pl.BlockSpec((_PPS, 8, _C), lambda b: (b, 0, 0))],
        out_shape=[jax.ShapeDtypeStruct((_B, _M, _C), jnp.float32),
                   jax.ShapeDtypeStruct((_B, 8, _C), jnp.float32)],
        compiler_params=pltpu.CompilerParams(
            dimension_semantics=("parallel",)),
    )(feat, bboxes)

    obj_flat, y, z = pl.pallas_call(
        _dense_kernel,
        grid=(2 * _NCB,),
        in_specs=[pl.BlockSpec((_B * _M, _C), lambda i: (0, 0)),
                  pl.BlockSpec((_B, 8, _C), lambda i: (0, 0, 0)),
                  pl.BlockSpec((_C, _CB), lambda i: (0, jnp.minimum(i, _NCB - 1))),
                  pl.BlockSpec((1, _CB), lambda i: (0, jnp.minimum(i, _NCB - 1))),
                  pl.BlockSpec((_C, _D), lambda i: (0, 0)),
                  pl.BlockSpec((1, _D), lambda i: (0, 0)),
                  pl.BlockSpec((2 * _D, _CB), lambda i: (0, jnp.maximum(i - _NCB, 0)))],
        out_specs=[pl.BlockSpec((_B * _M, _CB), lambda i: (0, jnp.minimum(i, _NCB - 1))),
                   pl.BlockSpec((_B * _M, _CB), lambda i: (0, jnp.maximum(i - _NCB, 0))),
                   pl.BlockSpec((_B, _CB), lambda i: (0, jnp.maximum(i - _NCB, 0)))],
        out_shape=[jax.ShapeDtypeStruct((_B * _M, _D), jnp.float32),
                   jax.ShapeDtypeStruct((_B * _M, _D), jnp.float32),
                   jax.ShapeDtypeStruct((_B, _D), jnp.float32)],
        scratch_shapes=[pltpu.VMEM((_B * _M, _D), jnp.float32),
                        pltpu.VMEM((_B, _D), jnp.float32)],
    )(pooled.reshape(_B * _M, _C), ctx8,
      W_obj, b_obj.reshape(1, _D), W_ctx, b_ctx.reshape(1, _D), W1)

    op = obj_pairs.astype(jnp.int32)
    idx = jnp.concatenate([op[..., 0].reshape(-1), op[..., 1].reshape(-1)])
    g = _pair_gather_sum(y, idx)

    f3 = pl.pallas_call(
        _mlp_kernel,
        grid=(_B // _BPS,),
        in_specs=[pl.BlockSpec((_BPS * _R, _D), lambda i: (i, 0)),
                  pl.BlockSpec((_B, _D), lambda i: (0, 0)),
                  pl.BlockSpec(memory_space=pltpu.SMEM),
                  pl.BlockSpec((1, _H1), lambda i: (0, 0)),
                  pl.BlockSpec((_H1, _H2), lambda i: (0, 0)),
                  pl.BlockSpec((1, _H2), lambda i: (0, 0))],
        out_specs=pl.BlockSpec((_BPS * _R, _H2), lambda i: (i, 0)),
        out_shape=jax.ShapeDtypeStruct((_B * _R, _H2), jnp.float32),
    )(g, z, num_rel, b1.reshape(1, _H1), W2, b2.reshape(1, _H2))

    return obj_flat, f3


# speedup vs baseline: 1.0610x; 1.0610x over previous
import functools

import jax
import jax.numpy as jnp
from jax import lax
from jax.experimental import pallas as pl
from jax.experimental.pallas import tpu as pltpu
from jax.experimental.pallas import tpu_sc as plsc

_B, _C, _H, _W = 16, 768, 14, 14
_M, _R = 32, 64
_D = 1024
_H1, _H2 = 1024, 512
_HW = _H * _W

_NW = 32

_PPS = 8


def _pool_kernel(feat_ref, bbox_ref, pooled_ref, ctx_ref):
    for k in range(_PPS):
        f = feat_ref[k]
        bb = bbox_ref[k]
        x0 = jnp.minimum(bb[:, 0:1], bb[:, 2:3])
        x1 = jnp.maximum(bb[:, 0:1], bb[:, 2:3])
        y0 = jnp.minimum(bb[:, 1:2], bb[:, 3:4])
        y1 = jnp.maximum(bb[:, 1:2], bb[:, 3:4])
        p = lax.broadcasted_iota(jnp.int32, (1, _HW), 1)
        xw = ((p % _W).astype(jnp.float32) + 0.5) / _W
        yh = ((p // _W).astype(jnp.float32) + 0.5) / _H
        mask = ((xw >= x0) & (xw <= x1) & (yh >= y0) & (yh <= y1))
        cnt = jnp.maximum(jnp.sum(mask.astype(jnp.float32), axis=1,
                                  keepdims=True), 1.0)
        row = lax.broadcasted_iota(jnp.int32, (8, _HW), 0)
        mext = jnp.concatenate([mask, row == 0], axis=0).astype(jnp.bfloat16)
        pooled = lax.dot_general(mext, f, (((1,), (1,)), ((), ())),
                                 preferred_element_type=jnp.float32)
        pooled_ref[k] = pooled[:_M] / cnt
        ctx_ref[k] = pooled[_M:] * (1.0 / _HW)


def _dense_kernel(pooled_ref, ctx8_ref, wobj_ref, bobj_ref, wctx_ref, bctx_ref,
                  w1_ref, obj_ref, y_ref, z_ref):
    obj = jnp.maximum(
        jnp.dot(pooled_ref[...], wobj_ref[...], preferred_element_type=jnp.float32)
        + bobj_ref[...], 0.0)
    obj_ref[...] = obj
    y_ref[...] = jnp.dot(obj, w1_ref[:_D], preferred_element_type=jnp.float32)
    ctx = jnp.maximum(
        jnp.dot(ctx8_ref[:, 0, :], wctx_ref[...],
                preferred_element_type=jnp.float32) + bctx_ref[...], 0.0)
    z_ref[...] = jnp.dot(ctx, w1_ref[_D:], preferred_element_type=jnp.float32)


_OUT = _B * _R
_OPW = _OUT // _NW


def _pair_gather_sum(y, idx):
    @functools.partial(
        pl.kernel,
        mesh=plsc.VectorSubcoreMesh(core_axis_name="c", subcore_axis_name="s"),
        out_type=jax.ShapeDtypeStruct((_OUT, _D), jnp.float32),
        scratch_types=[
            pltpu.VMEM((_OPW,), jnp.int32),
            pltpu.VMEM((_OPW,), jnp.int32),
            pltpu.VMEM((_OPW, _D), jnp.float32),
            pltpu.VMEM((_OPW, _D), jnp.float32),
            pltpu.SemaphoreType.DMA,
        ],
    )
    def k(table_hbm, idx_hbm, out_hbm, idx0_v, idx1_v, buf0, buf1, sem):
        wid = lax.axis_index("s") * 2 + lax.axis_index("c")
        base = wid * _OPW
        pltpu.sync_copy(idx_hbm.at[pl.ds(base, _OPW)], idx0_v)
        pltpu.sync_copy(idx_hbm.at[pl.ds(_OUT + base, _OPW)], idx1_v)
        off = (wid // (_NW // _B)) * _M
        for j in range(_OPW // 16):
            s = pl.ds(j * 16, 16)
            idx0_v[s] = idx0_v[s] + off
            idx1_v[s] = idx1_v[s] + off
        cp0 = pltpu.async_copy(table_hbm.at[idx0_v], buf0, sem)
        cp1 = pltpu.async_copy(table_hbm.at[idx1_v], buf1, sem)
        cp0.wait()
        cp1.wait()

        def body(r, carry):
            for c in range(_D // 16):
                s = pl.ds(c * 16, 16)
                buf0[r, s] = buf0[r, s] + buf1[r, s]
            return carry

        lax.fori_loop(0, _OPW, body, 0)
        pltpu.sync_copy(buf0, out_hbm.at[pl.ds(base, _OPW)])

    return k(y, idx)


_BPS = 8


def _mlp_kernel(g_ref, z_ref, nrel_ref, b1_ref, w2_ref, b2_ref, out_ref):
    i = pl.program_id(0)
    for k in range(_BPS):
        b = i * _BPS + k
        nr = nrel_ref[b]
        valid = (lax.broadcasted_iota(jnp.int32, (_R, 1), 0) < nr).astype(jnp.float32)
        rows = g_ref[pl.ds(k * _R, _R), :]
        zrow = z_ref[pl.ds(b, 1), :]
        h = jnp.maximum(valid * (0.5 * rows + zrow) + b1_ref[...], 0.0)
        out_ref[pl.ds(k * _R, _R), :] = jnp.maximum(
            jnp.dot(h, w2_ref[...], preferred_element_type=jnp.float32)
            + b2_ref[...], 0.0)


def kernel(frame_deep_features, bboxes, num_obj, obj_pairs, num_rel,
           W_obj, b_obj, W_ctx, b_ctx, W1, b1, W2, b2):
    feat = frame_deep_features.reshape(_B, _C, _HW).astype(jnp.bfloat16)
    pooled, ctx8 = pl.pallas_call(
        _pool_kernel,
        grid=(_B // _PPS,),
        in_specs=[pl.BlockSpec((_PPS, _C, _HW), lambda b: (b, 0, 0)),
                  pl.BlockSpec((_PPS, _M, 4), lambda b: (b, 0, 0))],
        out_specs=[pl.BlockSpec((_PPS, _M, _C), lambda b: (b, 0, 0)),
                   pl.BlockSpec((_PPS, 8, _C), lambda b: (b, 0, 0))],
        out_shape=[jax.ShapeDtypeStruct((_B, _M, _C), jnp.float32),
                   jax.ShapeDtypeStruct((_B, 8, _C), jnp.float32)],
        compiler_params=pltpu.CompilerParams(
            dimension_semantics=("parallel",)),
    )(feat, bboxes)

    obj_flat, y, z = pl.pallas_call(
        _dense_kernel,
        out_shape=[jax.ShapeDtypeStruct((_B * _M, _D), jnp.float32),
                   jax.ShapeDtypeStruct((_B * _M, _D), jnp.float32),
                   jax.ShapeDtypeStruct((_B, _D), jnp.float32)],
    )(pooled.reshape(_B * _M, _C), ctx8,
      W_obj, b_obj.reshape(1, _D), W_ctx, b_ctx.reshape(1, _D), W1)

    op = obj_pairs.astype(jnp.int32)
    idx = jnp.concatenate([op[..., 0].reshape(-1), op[..., 1].reshape(-1)])
    g = _pair_gather_sum(y, idx)

    f3 = pl.pallas_call(
        _mlp_kernel,
        grid=(_B // _BPS,),
        in_specs=[pl.BlockSpec((_BPS * _R, _D), lambda i: (i, 0)),
                  pl.BlockSpec((_B, _D), lambda i: (0, 0)),
                  pl.BlockSpec(memory_space=pltpu.SMEM),
                  pl.BlockSpec((1, _H1), lambda i: (0, 0)),
                  pl.BlockSpec((_H1, _H2), lambda i: (0, 0)),
                  pl.BlockSpec((1, _H2), lambda i: (0, 0))],
        out_specs=pl.BlockSpec((_BPS * _R, _H2), lambda i: (i, 0)),
        out_shape=jax.ShapeDtypeStruct((_B * _R, _H2), jnp.float32),
    )(g, z, num_rel, b1.reshape(1, _H1), W2, b2.reshape(1, _H2))

    return obj_flat, f3
